# baseline (speedup 1.0000x reference)
import jax
import jax.numpy as jnp
from jax import lax
from jax.experimental import pallas as pl
from jax.experimental.pallas import tpu as pltpu
from jax.experimental.pallas import tpu_sc as plsc

NUM_CLASSES = 100
ROWS = 16384 * 50
NUM_CORES = 2
NUM_SUBCORES = 16
NW = NUM_CORES * NUM_SUBCORES
ROWS_PER_W = ROWS // NW
CH = 256
N_CHUNKS = ROWS_PER_W // CH
CHW = CH * NUM_CLASSES
LANES = 16


def _onehot_body(idx_hbm, out_hbm, idx_v, buf0, buf1, sem0, sem1):
    wid = lax.axis_index("s") * NUM_CORES + lax.axis_index("c")
    base = wid * ROWS_PER_W

    pltpu.sync_copy(idx_hbm.at[pl.ds(base, ROWS_PER_W)], idx_v)

    lane100 = lax.iota(jnp.int32, LANES) * NUM_CLASSES
    ones16 = jnp.full((LANES,), 1, jnp.int32)
    zeros16 = jnp.zeros((LANES,), jnp.int32)

    def _zero(k, carry):
        buf0[pl.ds(k * LANES, LANES)] = zeros16
        buf1[pl.ds(k * LANES, LANES)] = zeros16
        return carry
    lax.fori_loop(0, CHW // LANES, _zero, 0)

    def _scatter(buf, c, vals):
        for i in range(CH // LANES):
            idxv = idx_v[pl.ds(c * CH + i * LANES, LANES)]
            pos = idxv + (lane100 + i * LANES * NUM_CLASSES)
            plsc.store_scatter(buf, [pos], vals)

    def _out_slice(c):
        return out_hbm.at[pl.ds(base * NUM_CLASSES + c * CHW, CHW)]

    for b, (buf, sem) in enumerate(((buf0, sem0), (buf1, sem1))):
        _scatter(buf, b, ones16)
        pltpu.async_copy(buf, _out_slice(b), sem)

    def _step(k, carry):
        cc = 2 * k
        for b, (buf, sem) in enumerate(((buf0, sem0), (buf1, sem1))):
            c = cc + b
            pltpu.make_async_copy(buf, _out_slice(c), sem).wait()
            _scatter(buf, c - 2, zeros16)
            _scatter(buf, c, ones16)
            pltpu.async_copy(buf, _out_slice(c), sem)
        return carry
    lax.fori_loop(1, N_CHUNKS // 2, _step, 0)

    pltpu.make_async_copy(buf0, _out_slice(N_CHUNKS - 2), sem0).wait()
    pltpu.make_async_copy(buf1, _out_slice(N_CHUNKS - 1), sem1).wait()


@jax.jit
def kernel(atom_type):
    idx_flat = atom_type.reshape(ROWS)
    mesh = plsc.VectorSubcoreMesh(core_axis_name="c", subcore_axis_name="s")
    out = pl.kernel(
        _onehot_body,
        out_type=jax.ShapeDtypeStruct((ROWS * NUM_CLASSES,), jnp.int32),
        mesh=mesh,
        compiler_params=pltpu.CompilerParams(needs_layout_passes=False),
        scratch_types=[
            pltpu.VMEM((ROWS_PER_W,), jnp.int32),
            pltpu.VMEM((CHW,), jnp.int32),
            pltpu.VMEM((CHW,), jnp.int32),
            pltpu.SemaphoreType.DMA,
            pltpu.SemaphoreType.DMA,
        ],
    )(idx_flat)
    return out.reshape(atom_type.shape[0], atom_type.shape[1], NUM_CLASSES)


# speedup vs baseline: 1.0003x; 1.0003x over previous
import jax
import jax.numpy as jnp
from jax import lax
from jax.experimental import pallas as pl
from jax.experimental.pallas import tpu as pltpu
from jax.experimental.pallas import tpu_sc as plsc

NUM_CLASSES = 100
ROWS = 16384 * 50
NUM_CORES = 2
NUM_SUBCORES = 16
NW = NUM_CORES * NUM_SUBCORES
ROWS_PER_W = ROWS // NW
CH = 256
N_CHUNKS = ROWS_PER_W // CH
CHW = CH * NUM_CLASSES
NBUF = 4
LANES = 16


def _onehot_body(idx_hbm, out_hbm, idx_v, buf0, buf1, buf2, buf3,
                 sem0, sem1, sem2, sem3):
    bufs = (buf0, buf1, buf2, buf3)
    sems = (sem0, sem1, sem2, sem3)
    wid = lax.axis_index("s") * NUM_CORES + lax.axis_index("c")
    base = wid * ROWS_PER_W

    pltpu.sync_copy(idx_hbm.at[pl.ds(base, ROWS_PER_W)], idx_v)

    lane100 = lax.iota(jnp.int32, LANES) * NUM_CLASSES
    ones16 = jnp.full((LANES,), 1, jnp.int32)
    zeros16 = jnp.zeros((LANES,), jnp.int32)

    def _zero(k, carry):
        for buf in bufs:
            buf[pl.ds(k * LANES, LANES)] = zeros16
        return carry
    lax.fori_loop(0, CHW // LANES, _zero, 0)

    def _scatter(buf, c, vals):
        for i in range(CH // LANES):
            idxv = idx_v[pl.ds(c * CH + i * LANES, LANES)]
            pos = idxv + (lane100 + i * LANES * NUM_CLASSES)
            plsc.store_scatter(buf, [pos], vals)

    def _out_slice(c):
        return out_hbm.at[pl.ds(base * NUM_CLASSES + c * CHW, CHW)]

    for b in range(NBUF):
        _scatter(bufs[b], b, ones16)
        pltpu.async_copy(bufs[b], _out_slice(b), sems[b])

    def _step(k, carry):
        cc = NBUF * k
        for b in range(NBUF):
            c = cc + b
            pltpu.make_async_copy(bufs[b], _out_slice(c), sems[b]).wait()
            _scatter(bufs[b], c - NBUF, zeros16)
            _scatter(bufs[b], c, ones16)
            pltpu.async_copy(bufs[b], _out_slice(c), sems[b])
        return carry
    lax.fori_loop(1, N_CHUNKS // NBUF, _step, 0)

    for b in range(NBUF):
        pltpu.make_async_copy(
            bufs[b], _out_slice(N_CHUNKS - NBUF + b), sems[b]).wait()


@jax.jit
def kernel(atom_type):
    idx_flat = atom_type.reshape(ROWS)
    mesh = plsc.VectorSubcoreMesh(core_axis_name="c", subcore_axis_name="s")
    out = pl.kernel(
        _onehot_body,
        out_type=jax.ShapeDtypeStruct((ROWS * NUM_CLASSES,), jnp.int32),
        mesh=mesh,
        compiler_params=pltpu.CompilerParams(needs_layout_passes=False),
        scratch_types=[
            pltpu.VMEM((ROWS_PER_W,), jnp.int32),
            pltpu.VMEM((CHW,), jnp.int32),
            pltpu.VMEM((CHW,), jnp.int32),
            pltpu.VMEM((CHW,), jnp.int32),
            pltpu.VMEM((CHW,), jnp.int32),
            pltpu.SemaphoreType.DMA,
            pltpu.SemaphoreType.DMA,
            pltpu.SemaphoreType.DMA,
            pltpu.SemaphoreType.DMA,
        ],
    )(idx_flat)
    return out.reshape(atom_type.shape[0], atom_type.shape[1], NUM_CLASSES)
